# Initial kernel scaffold; baseline (speedup 1.0000x reference)
import jax
import jax.numpy as jnp
from jax.experimental import pallas as pl


def kernel(x, edge_index, W1, b1, W2, b2, W3, b3, W4, b4, W5, b5, Wfc, bfc):
    raise NotImplementedError("write your pallas kernel here")


# speedup vs baseline: 2.8190x; 2.8190x over previous
import functools

import jax
import jax.numpy as jnp
from jax import lax
from jax.experimental import pallas as pl
from jax.experimental.pallas import tpu as pltpu
from jax.experimental.pallas import tpu_sc as plsc

NC = 2
NS = 16
NW = NC * NS
L = 16

N = 10000
NP = 10240
K = 128
KE = 64
EO_LEN = 72

_MESH = plsc.VectorSubcoreMesh(core_axis_name="c", subcore_axis_name="s")
_SC_PARAMS = pltpu.CompilerParams(needs_layout_passes=False)


def _splat_f(buf, k):
    return plsc.load_gather(buf, [jnp.full((L,), k, jnp.int32)])


def _scalar_i(buf, i):
    return jnp.max(plsc.load_gather(buf, [jnp.full((L,), i, jnp.int32)]))


def _rsqrt16(x):
    i = lax.bitcast_convert_type(x, jnp.int32)
    i = jnp.int32(0x5F3759DF) - (i >> 1)
    y = lax.bitcast_convert_type(i, jnp.float32)
    for _ in range(3):
        y = y * (1.5 - 0.5 * x * y * y)
    return y


def _iota16():
    return lax.iota(jnp.int32, L)


def _wid():
    return lax.axis_index("s") * NC + lax.axis_index("c")


def _make_degree():
    nw_nodes = NP // NW

    @functools.partial(
        pl.kernel,
        out_type=(
            jax.ShapeDtypeStruct((NP,), jnp.float32),
            jax.ShapeDtypeStruct((NP,), jnp.float32),
        ),
        mesh=_MESH,
        compiler_params=_SC_PARAMS,
        scratch_types=[
            pltpu.VMEM((nw_nodes,), jnp.float32),
            pltpu.VMEM((nw_nodes,), jnp.float32),
            pltpu.VMEM((K,), jnp.int32),
            pltpu.VMEM((EO_LEN,), jnp.int32),
        ],
    )
    def degree(dst_hbm, eo_hbm, dinv_hbm, selfw_hbm, deg_t, sw_t, dbuf, eo_v):
        wid = _wid()
        base_n = wid * nw_nodes
        pltpu.sync_copy(eo_hbm, eo_v)

        for j in range(nw_nodes // L):
            deg_t[pl.ds(j * L, L)] = jnp.full((L,), 1.0, jnp.float32)

        my_lo = _scalar_i(eo_v, wid)
        my_hi = _scalar_i(eo_v, wid + 1)
        a0 = (my_lo // 8) * 8
        nb = jnp.maximum((my_hi - a0 + K - 1) // K, 0)

        def batch(bi, _):
            a = pl.multiple_of(a0 + bi * K, 8)
            pltpu.sync_copy(dst_hbm.at[pl.ds(a, K)], dbuf)

            def onehot(k, _):
                eid = a + k
                valid = (eid >= my_lo) & (eid < my_hi)
                lidx = _scalar_i(dbuf, k) - base_n
                lidx = jnp.minimum(jnp.maximum(lidx, 0), nw_nodes - 1)
                row = (lidx // L) * L
                inc = jnp.where((_iota16() == lidx - row)
                                & jnp.full((L,), valid, jnp.bool_), 1.0, 0.0)
                deg_t[pl.ds(row, L)] = deg_t[pl.ds(row, L)] + inc
                return 0
            lax.fori_loop(0, K, onehot, 0)
            return 0
        lax.fori_loop(0, nb, batch, 0)

        for j in range(nw_nodes // L):
            d = deg_t[pl.ds(j * L, L)]
            node = jnp.full((L,), base_n + j * L, jnp.int32) + _iota16()
            real = node < N
            deg_t[pl.ds(j * L, L)] = jnp.where(real, _rsqrt16(d), 0.0)
            sw_t[pl.ds(j * L, L)] = jnp.where(real, 1.0 / d, 0.0)
        pltpu.sync_copy(deg_t, dinv_hbm.at[pl.ds(base_n, nw_nodes)])
        pltpu.sync_copy(sw_t, selfw_hbm.at[pl.ds(base_n, nw_nodes)])

    return degree


def _make_norm(Ep):
    ebw = Ep // NW

    @functools.partial(
        pl.kernel,
        out_type=jax.ShapeDtypeStruct((Ep,), jnp.float32),
        mesh=_MESH,
        compiler_params=_SC_PARAMS,
        scratch_types=[
            pltpu.VMEM((NP,), jnp.float32),
            pltpu.VMEM((K,), jnp.int32),
            pltpu.VMEM((K,), jnp.int32),
            pltpu.VMEM((K,), jnp.float32),
        ],
    )
    def norm(src_hbm, dst_hbm, dinv_hbm, nrm_hbm, dinv_v, sbuf, dbuf, fbuf):
        wid = _wid()
        pltpu.sync_copy(dinv_hbm, dinv_v)

        def batch(bi, _):
            a = pl.multiple_of(wid * ebw + bi * K, 8)
            pltpu.sync_copy(src_hbm.at[pl.ds(a, K)], sbuf)
            pltpu.sync_copy(dst_hbm.at[pl.ds(a, K)], dbuf)
            for j in range(K // L):
                si = sbuf[pl.ds(j * L, L)]
                di = dbuf[pl.ds(j * L, L)]
                fbuf[pl.ds(j * L, L)] = (plsc.load_gather(dinv_v, [si])
                                         * plsc.load_gather(dinv_v, [di]))
            pltpu.sync_copy(fbuf, nrm_hbm.at[pl.ds(a, K)])
            return 0
        lax.fori_loop(0, ebw // K, batch, 0)

    return norm


def _make_propagate(D, C, Ep):
    Nc = NP // C
    passes = C // NW

    @functools.partial(
        pl.kernel,
        out_type=jax.ShapeDtypeStruct((NP, D), jnp.float32),
        mesh=_MESH,
        compiler_params=_SC_PARAMS,
        scratch_types=[
            pltpu.VMEM((Nc + 1, D), jnp.float32),
            pltpu.VMEM((KE,), jnp.int32),
            pltpu.VMEM((KE,), jnp.int32),
            pltpu.VMEM((KE,), jnp.float32),
            pltpu.VMEM((KE, D), jnp.float32),
            pltpu.VMEM((Nc,), jnp.float32),
            pltpu.VMEM((EO_LEN,), jnp.int32),
            pltpu.SemaphoreType.DMA,
        ],
    )
    def propagate(h_hbm, src_hbm, dst_hbm, nrm_hbm, selfw_hbm, eo_hbm, out_hbm,
                  acc, sbuf, dbuf, nbuf, rows, swbuf, eo_v, gsem):
        wid = _wid()
        pltpu.sync_copy(eo_hbm, eo_v)

        for p in range(passes):
            chunk = p * NW + wid
            base = chunk * Nc

            pltpu.sync_copy(h_hbm.at[pl.ds(base, Nc)], acc.at[pl.ds(0, Nc)])
            pltpu.sync_copy(selfw_hbm.at[pl.ds(base, Nc)], swbuf)

            def initk(k, _):
                w = _splat_f(swbuf, k)
                for j in range(D // L):
                    acc[k, pl.ds(j * L, L)] = acc[k, pl.ds(j * L, L)] * w
                return 0
            lax.fori_loop(0, Nc, initk, 0)

            my_lo = _scalar_i(eo_v, chunk)
            my_hi = _scalar_i(eo_v, chunk + 1)
            a0 = (my_lo // 8) * 8
            nb = jnp.maximum((my_hi - a0 + KE - 1) // KE, 0)

            def batch(bi, _):
                a = pl.multiple_of(a0 + bi * KE, 8)
                pltpu.sync_copy(src_hbm.at[pl.ds(a, KE)], sbuf)
                pltpu.sync_copy(dst_hbm.at[pl.ds(a, KE)], dbuf)
                pltpu.sync_copy(nrm_hbm.at[pl.ds(a, KE)], nbuf)
                for j in range(KE // L):
                    eid = jnp.full((L,), a + j * L, jnp.int32) + _iota16()
                    ok = (eid >= my_lo) & (eid < my_hi)
                    nv = nbuf[pl.ds(j * L, L)]
                    nbuf[pl.ds(j * L, L)] = jnp.where(ok, nv, 0.0)
                    dv = dbuf[pl.ds(j * L, L)] - base
                    dv = jnp.minimum(jnp.maximum(dv, 0), Nc)
                    dbuf[pl.ds(j * L, L)] = dv
                pltpu.async_copy(h_hbm.at[sbuf], rows, gsem).wait()

                def scalek(k, _):
                    w = _splat_f(nbuf, k)
                    lidx = _scalar_i(dbuf, k)
                    for j in range(D // L):
                        plsc.addupdate(acc.at[lidx, pl.ds(j * L, L)],
                                       rows[k, pl.ds(j * L, L)] * w)
                    return 0
                lax.fori_loop(0, KE, scalek, 0)
                return 0
            lax.fori_loop(0, nb, batch, 0)

            pltpu.sync_copy(acc.at[pl.ds(0, Nc)], out_hbm.at[pl.ds(base, Nc)])

    return propagate


def _matmul(x, w, b_in=None, b_out=None, block_m=1024):
    m, d_in = x.shape
    d_out = w.shape[1]
    relu_in = b_in is not None
    add_out = b_out is not None

    def kern(x_ref, w_ref, bi_ref, bo_ref, o_ref):
        a = x_ref[...]
        if relu_in:
            a = jnp.maximum(a + bi_ref[...], 0.0)
        acc = jnp.dot(a, w_ref[...], preferred_element_type=jnp.float32)
        if add_out:
            acc = acc + bo_ref[...]
        o_ref[...] = acc

    bi = b_in.reshape(1, d_in) if relu_in else jnp.zeros((1, d_in), jnp.float32)
    bo = b_out.reshape(1, d_out) if add_out else jnp.zeros((1, d_out), jnp.float32)
    return pl.pallas_call(
        kern,
        grid=(m // block_m,),
        in_specs=[
            pl.BlockSpec((block_m, d_in), lambda i: (i, 0)),
            pl.BlockSpec((d_in, d_out), lambda i: (0, 0)),
            pl.BlockSpec((1, d_in), lambda i: (0, 0)),
            pl.BlockSpec((1, d_out), lambda i: (0, 0)),
        ],
        out_specs=pl.BlockSpec((block_m, d_out), lambda i: (i, 0)),
        out_shape=jax.ShapeDtypeStruct((m, d_out), jnp.float32),
    )(x, w, bi, bo)


@jax.jit
def kernel(x, edge_index, W1, b1, W2, b2, W3, b3, W4, b4, W5, b5, Wfc, bfc):
    E = edge_index.shape[1]
    Ep = ((E + NW * K) // (NW * K)) * (NW * K)

    src = edge_index[0].astype(jnp.int32)
    dst = edge_index[1].astype(jnp.int32)
    perm = jnp.argsort(dst)
    src_s = jnp.concatenate([src[perm], jnp.zeros((Ep - E,), jnp.int32)])
    dst_s = jnp.concatenate([dst[perm], jnp.full((Ep - E,), N, jnp.int32)])

    def edge_offsets(C):
        bounds = jnp.array([min(i * (NP // C), N) for i in range(C + 1)]
                           + [0] * (EO_LEN - C - 1), jnp.int32)
        return jnp.searchsorted(dst_s, bounds).astype(jnp.int32)

    eo32 = edge_offsets(NW)
    eo64 = edge_offsets(2 * NW)

    dinv, selfw = _make_degree()(dst_s, eo32)
    nrm = _make_norm(Ep)(src_s, dst_s, dinv)

    xp = jnp.pad(x, ((0, NP - N), (0, 0)))
    W5p = jnp.pad(W5, ((0, 0), (0, 118)))
    b5p = jnp.pad(b5, (0, 118))
    Wfcp = jnp.pad(Wfc, ((0, 118), (0, 125)))
    bfcp = jnp.pad(bfc, (0, 125))

    prop512 = _make_propagate(512, 2 * NW, Ep)
    prop256 = _make_propagate(256, NW, Ep)
    prop128 = _make_propagate(128, NW, Ep)

    h = _matmul(xp, W1)
    p = prop512(h, src_s, dst_s, nrm, selfw, eo64)
    h = _matmul(p, W2, b_in=b1)
    p = prop512(h, src_s, dst_s, nrm, selfw, eo64)
    h = _matmul(p, W3, b_in=b2)
    p = prop256(h, src_s, dst_s, nrm, selfw, eo32)
    h = _matmul(p, W4, b_in=b3)
    p = prop128(h, src_s, dst_s, nrm, selfw, eo32)
    h = _matmul(p, W5p, b_in=b4)
    p = prop128(h, src_s, dst_s, nrm, selfw, eo32)
    out = _matmul(p, Wfcp, b_in=b5p, b_out=bfcp)
    return out[:N, :3]


# speedup vs baseline: 3.0845x; 1.0942x over previous
import functools

import jax
import jax.numpy as jnp
from jax import lax
from jax.experimental import pallas as pl
from jax.experimental.pallas import tpu as pltpu
from jax.experimental.pallas import tpu_sc as plsc

NC = 2
NS = 16
NW = NC * NS
L = 16

N = 10000
NP = 10240
K = 128
KE = 64
EO_LEN = 136

_MESH = plsc.VectorSubcoreMesh(core_axis_name="c", subcore_axis_name="s")
_SC_PARAMS = pltpu.CompilerParams(needs_layout_passes=False)


def _splat_f(buf, k):
    return plsc.load_gather(buf, [jnp.full((L,), k, jnp.int32)])


def _scalar_i(buf, i):
    return jnp.max(plsc.load_gather(buf, [jnp.full((L,), i, jnp.int32)]))


def _rsqrt16(x):
    i = lax.bitcast_convert_type(x, jnp.int32)
    i = jnp.int32(0x5F3759DF) - (i >> 1)
    y = lax.bitcast_convert_type(i, jnp.float32)
    for _ in range(3):
        y = y * (1.5 - 0.5 * x * y * y)
    return y


def _iota16():
    return lax.iota(jnp.int32, L)


def _wid():
    return lax.axis_index("s") * NC + lax.axis_index("c")


def _make_degree():
    nw_nodes = NP // NW

    @functools.partial(
        pl.kernel,
        out_type=(
            jax.ShapeDtypeStruct((NP,), jnp.float32),
            jax.ShapeDtypeStruct((NP,), jnp.float32),
        ),
        mesh=_MESH,
        compiler_params=_SC_PARAMS,
        scratch_types=[
            pltpu.VMEM((nw_nodes,), jnp.float32),
            pltpu.VMEM((nw_nodes,), jnp.float32),
            pltpu.VMEM((K,), jnp.int32),
            pltpu.VMEM((EO_LEN,), jnp.int32),
        ],
    )
    def degree(dst_hbm, eo_hbm, dinv_hbm, selfw_hbm, deg_t, sw_t, dbuf, eo_v):
        wid = _wid()
        base_n = wid * nw_nodes
        pltpu.sync_copy(eo_hbm, eo_v)

        for j in range(nw_nodes // L):
            deg_t[pl.ds(j * L, L)] = jnp.full((L,), 1.0, jnp.float32)

        my_lo = _scalar_i(eo_v, wid)
        my_hi = _scalar_i(eo_v, wid + 1)
        a0 = (my_lo // 8) * 8
        nb = jnp.maximum((my_hi - a0 + K - 1) // K, 0)

        def batch(bi, _):
            a = pl.multiple_of(a0 + bi * K, 8)
            pltpu.sync_copy(dst_hbm.at[pl.ds(a, K)], dbuf)

            def onehot(k, _):
                eid = a + k
                valid = (eid >= my_lo) & (eid < my_hi)
                lidx = _scalar_i(dbuf, k) - base_n
                lidx = jnp.minimum(jnp.maximum(lidx, 0), nw_nodes - 1)
                row = (lidx // L) * L
                inc = jnp.where((_iota16() == lidx - row)
                                & jnp.full((L,), valid, jnp.bool_), 1.0, 0.0)
                deg_t[pl.ds(row, L)] = deg_t[pl.ds(row, L)] + inc
                return 0
            lax.fori_loop(0, K, onehot, 0)
            return 0
        lax.fori_loop(0, nb, batch, 0)

        for j in range(nw_nodes // L):
            d = deg_t[pl.ds(j * L, L)]
            node = jnp.full((L,), base_n + j * L, jnp.int32) + _iota16()
            real = node < N
            deg_t[pl.ds(j * L, L)] = jnp.where(real, _rsqrt16(d), 0.0)
            sw_t[pl.ds(j * L, L)] = jnp.where(real, 1.0 / d, 0.0)
        pltpu.sync_copy(deg_t, dinv_hbm.at[pl.ds(base_n, nw_nodes)])
        pltpu.sync_copy(sw_t, selfw_hbm.at[pl.ds(base_n, nw_nodes)])

    return degree


def _make_norm(Ep):
    ebw = Ep // NW

    @functools.partial(
        pl.kernel,
        out_type=jax.ShapeDtypeStruct((Ep,), jnp.float32),
        mesh=_MESH,
        compiler_params=_SC_PARAMS,
        scratch_types=[
            pltpu.VMEM((NP,), jnp.float32),
            pltpu.VMEM((K,), jnp.int32),
            pltpu.VMEM((K,), jnp.int32),
            pltpu.VMEM((K,), jnp.float32),
        ],
    )
    def norm(src_hbm, dst_hbm, dinv_hbm, nrm_hbm, dinv_v, sbuf, dbuf, fbuf):
        wid = _wid()
        pltpu.sync_copy(dinv_hbm, dinv_v)

        def batch(bi, _):
            a = pl.multiple_of(wid * ebw + bi * K, 8)
            pltpu.sync_copy(src_hbm.at[pl.ds(a, K)], sbuf)
            pltpu.sync_copy(dst_hbm.at[pl.ds(a, K)], dbuf)
            for j in range(K // L):
                si = sbuf[pl.ds(j * L, L)]
                di = dbuf[pl.ds(j * L, L)]
                fbuf[pl.ds(j * L, L)] = (plsc.load_gather(dinv_v, [si])
                                         * plsc.load_gather(dinv_v, [di]))
            pltpu.sync_copy(fbuf, nrm_hbm.at[pl.ds(a, K)])
            return 0
        lax.fori_loop(0, ebw // K, batch, 0)

    return norm


def _make_propagate(D, C, Ep):
    Nc = NP // C
    passes = C // NW

    @functools.partial(
        pl.kernel,
        out_type=jax.ShapeDtypeStruct((NP, D), jnp.float32),
        mesh=_MESH,
        compiler_params=_SC_PARAMS,
        scratch_types=[
            pltpu.VMEM((Nc + 1, D), jnp.float32),
            pltpu.VMEM((KE,), jnp.int32),
            pltpu.VMEM((KE,), jnp.int32),
            pltpu.VMEM((KE,), jnp.float32),
            pltpu.VMEM((KE,), jnp.int32),
            pltpu.VMEM((KE,), jnp.int32),
            pltpu.VMEM((KE,), jnp.float32),
            pltpu.VMEM((KE, D), jnp.float32),
            pltpu.VMEM((KE, D), jnp.float32),
            pltpu.VMEM((Nc,), jnp.float32),
            pltpu.VMEM((EO_LEN,), jnp.int32),
            pltpu.SemaphoreType.DMA,
            pltpu.SemaphoreType.DMA,
        ],
    )
    def propagate(h_hbm, src_hbm, dst_hbm, nrm_hbm, selfw_hbm, eo_hbm, out_hbm,
                  acc, sb0, db0, nb0, sb1, db1, nb1, rows0, rows1,
                  swbuf, eo_v, gsem0, gsem1):
        wid = _wid()
        pltpu.sync_copy(eo_hbm, eo_v)

        for p in range(passes):
            chunk = p * NW + wid
            base = chunk * Nc

            pltpu.sync_copy(h_hbm.at[pl.ds(base, Nc)], acc.at[pl.ds(0, Nc)])
            pltpu.sync_copy(selfw_hbm.at[pl.ds(base, Nc)], swbuf)

            def initk(k, _):
                w = _splat_f(swbuf, k)
                for j in range(D // L):
                    acc[k, pl.ds(j * L, L)] = acc[k, pl.ds(j * L, L)] * w
                return 0
            lax.fori_loop(0, Nc, initk, 0)

            my_lo = _scalar_i(eo_v, chunk)
            my_hi = _scalar_i(eo_v, chunk + 1)
            a0 = (my_lo // 8) * 8
            nb = jnp.maximum((my_hi - a0 + KE - 1) // KE, 0)
            nb2 = (nb + 1) // 2

            def addr(bi):
                return pl.multiple_of(
                    a0 + jnp.minimum(bi, (Ep - a0) // KE - 1) * KE, 8)

            def fetch(a, sb, db, nbf, rows_, sem):
                pltpu.sync_copy(src_hbm.at[pl.ds(a, KE)], sb)
                pltpu.sync_copy(dst_hbm.at[pl.ds(a, KE)], db)
                pltpu.sync_copy(nrm_hbm.at[pl.ds(a, KE)], nbf)
                pltpu.async_copy(h_hbm.at[sb], rows_, sem)

            def drain(rows_, sem):
                pltpu.make_async_copy(h_hbm.at[pl.ds(0, KE)], rows_, sem).wait()

            def consume(a, sb, db, nbf, rows_):
                for j in range(KE // L):
                    eid = jnp.full((L,), a + j * L, jnp.int32) + _iota16()
                    ok = (eid >= my_lo) & (eid < my_hi)
                    nv = nbf[pl.ds(j * L, L)]
                    nbf[pl.ds(j * L, L)] = jnp.where(ok, nv, 0.0)
                    dv = db[pl.ds(j * L, L)] - base
                    dv = jnp.minimum(jnp.maximum(dv, 0), Nc)
                    db[pl.ds(j * L, L)] = dv

                def scalek(k, _):
                    w = _splat_f(nbf, k)
                    lidx = _scalar_i(db, k)
                    for j in range(D // L):
                        plsc.addupdate(acc.at[lidx, pl.ds(j * L, L)],
                                       rows_[k, pl.ds(j * L, L)] * w)
                    return 0
                lax.fori_loop(0, KE, scalek, 0)

            fetch(addr(0), sb0, db0, nb0, rows0, gsem0)

            def pair(i, _):
                a_odd = addr(2 * i + 1)
                fetch(a_odd, sb1, db1, nb1, rows1, gsem1)
                drain(rows0, gsem0)
                consume(addr(2 * i), sb0, db0, nb0, rows0)
                fetch(addr(2 * i + 2), sb0, db0, nb0, rows0, gsem0)
                drain(rows1, gsem1)
                consume(a_odd, sb1, db1, nb1, rows1)
                return 0
            lax.fori_loop(0, nb2, pair, 0)
            drain(rows0, gsem0)

            pltpu.sync_copy(acc.at[pl.ds(0, Nc)], out_hbm.at[pl.ds(base, Nc)])

    return propagate


def _matmul(x, w, b_in=None, b_out=None, block_m=1024):
    m, d_in = x.shape
    d_out = w.shape[1]
    relu_in = b_in is not None
    add_out = b_out is not None

    def kern(x_ref, w_ref, bi_ref, bo_ref, o_ref):
        a = x_ref[...]
        if relu_in:
            a = jnp.maximum(a + bi_ref[...], 0.0)
        acc = jnp.dot(a, w_ref[...], preferred_element_type=jnp.float32)
        if add_out:
            acc = acc + bo_ref[...]
        o_ref[...] = acc

    bi = b_in.reshape(1, d_in) if relu_in else jnp.zeros((1, d_in), jnp.float32)
    bo = b_out.reshape(1, d_out) if add_out else jnp.zeros((1, d_out), jnp.float32)
    return pl.pallas_call(
        kern,
        grid=(m // block_m,),
        in_specs=[
            pl.BlockSpec((block_m, d_in), lambda i: (i, 0)),
            pl.BlockSpec((d_in, d_out), lambda i: (0, 0)),
            pl.BlockSpec((1, d_in), lambda i: (0, 0)),
            pl.BlockSpec((1, d_out), lambda i: (0, 0)),
        ],
        out_specs=pl.BlockSpec((block_m, d_out), lambda i: (i, 0)),
        out_shape=jax.ShapeDtypeStruct((m, d_out), jnp.float32),
    )(x, w, bi, bo)


@jax.jit
def kernel(x, edge_index, W1, b1, W2, b2, W3, b3, W4, b4, W5, b5, Wfc, bfc):
    E = edge_index.shape[1]
    Ep = ((E + NW * K) // (NW * K)) * (NW * K)

    src = edge_index[0].astype(jnp.int32)
    dst = edge_index[1].astype(jnp.int32)
    perm = jnp.argsort(dst)
    src_s = jnp.concatenate([src[perm], jnp.zeros((Ep - E,), jnp.int32)])
    dst_s = jnp.concatenate([dst[perm], jnp.full((Ep - E,), N, jnp.int32)])

    def edge_offsets(C):
        bounds = jnp.array([min(i * (NP // C), N) for i in range(C + 1)]
                           + [0] * (EO_LEN - C - 1), jnp.int32)
        return jnp.searchsorted(dst_s, bounds).astype(jnp.int32)

    eo32 = edge_offsets(NW)
    eo64 = edge_offsets(2 * NW)
    eo128 = edge_offsets(4 * NW)

    dinv, selfw = _make_degree()(dst_s, eo32)
    nrm = _make_norm(Ep)(src_s, dst_s, dinv)

    xp = jnp.pad(x, ((0, NP - N), (0, 0)))
    W5p = jnp.pad(W5, ((0, 0), (0, 118)))
    b5p = jnp.pad(b5, (0, 118))
    Wfcp = jnp.pad(Wfc, ((0, 118), (0, 125)))
    bfcp = jnp.pad(bfc, (0, 125))

    prop512 = _make_propagate(512, 4 * NW, Ep)
    prop256 = _make_propagate(256, 2 * NW, Ep)
    prop128 = _make_propagate(128, NW, Ep)

    h = _matmul(xp, W1)
    p = prop512(h, src_s, dst_s, nrm, selfw, eo128)
    h = _matmul(p, W2, b_in=b1)
    p = prop512(h, src_s, dst_s, nrm, selfw, eo128)
    h = _matmul(p, W3, b_in=b2)
    p = prop256(h, src_s, dst_s, nrm, selfw, eo64)
    h = _matmul(p, W4, b_in=b3)
    p = prop128(h, src_s, dst_s, nrm, selfw, eo32)
    h = _matmul(p, W5p, b_in=b4)
    p = prop128(h, src_s, dst_s, nrm, selfw, eo32)
    out = _matmul(p, Wfcp, b_in=b5p, b_out=bfcp)
    return out[:N, :3]


# speedup vs baseline: 3.3569x; 1.0883x over previous
import functools

import jax
import jax.numpy as jnp
from jax import lax
from jax.experimental import pallas as pl
from jax.experimental.pallas import tpu as pltpu
from jax.experimental.pallas import tpu_sc as plsc

NC = 2
NS = 16
NW = NC * NS
L = 16

N = 10000
NP = 10240
K = 128
KE = 64
EO_LEN = 136

_MESH = plsc.VectorSubcoreMesh(core_axis_name="c", subcore_axis_name="s")
_SC_PARAMS = pltpu.CompilerParams(needs_layout_passes=False)


def _splat_f(buf, k):
    return plsc.load_gather(buf, [jnp.full((L,), k, jnp.int32)])


def _scalar_i(buf, i):
    return jnp.max(plsc.load_gather(buf, [jnp.full((L,), i, jnp.int32)]))


def _rsqrt16(x):
    i = lax.bitcast_convert_type(x, jnp.int32)
    i = jnp.int32(0x5F3759DF) - (i >> 1)
    y = lax.bitcast_convert_type(i, jnp.float32)
    for _ in range(3):
        y = y * (1.5 - 0.5 * x * y * y)
    return y


def _iota16():
    return lax.iota(jnp.int32, L)


def _wid():
    return lax.axis_index("s") * NC + lax.axis_index("c")


def _make_degree():
    nw_nodes = NP // NW

    @functools.partial(
        pl.kernel,
        out_type=(
            jax.ShapeDtypeStruct((NP,), jnp.float32),
            jax.ShapeDtypeStruct((NP,), jnp.float32),
        ),
        mesh=_MESH,
        compiler_params=_SC_PARAMS,
        scratch_types=[
            pltpu.VMEM((nw_nodes,), jnp.float32),
            pltpu.VMEM((nw_nodes,), jnp.float32),
            pltpu.VMEM((K,), jnp.int32),
            pltpu.VMEM((EO_LEN,), jnp.int32),
        ],
    )
    def degree(dst_hbm, eo_hbm, dinv_hbm, selfw_hbm, deg_t, sw_t, dbuf, eo_v):
        wid = _wid()
        base_n = wid * nw_nodes
        pltpu.sync_copy(eo_hbm, eo_v)

        for j in range(nw_nodes // L):
            deg_t[pl.ds(j * L, L)] = jnp.full((L,), 1.0, jnp.float32)

        my_lo = _scalar_i(eo_v, wid)
        my_hi = _scalar_i(eo_v, wid + 1)
        a0 = (my_lo // 8) * 8
        nb = jnp.maximum((my_hi - a0 + K - 1) // K, 0)

        def batch(bi, _):
            a = pl.multiple_of(a0 + bi * K, 8)
            pltpu.sync_copy(dst_hbm.at[pl.ds(a, K)], dbuf)

            def onehot(k, _):
                eid = a + k
                valid = (eid >= my_lo) & (eid < my_hi)
                lidx = _scalar_i(dbuf, k) - base_n
                lidx = jnp.minimum(jnp.maximum(lidx, 0), nw_nodes - 1)
                row = (lidx // L) * L
                inc = jnp.where((_iota16() == lidx - row)
                                & jnp.full((L,), valid, jnp.bool_), 1.0, 0.0)
                deg_t[pl.ds(row, L)] = deg_t[pl.ds(row, L)] + inc
                return 0
            lax.fori_loop(0, K, onehot, 0)
            return 0
        lax.fori_loop(0, nb, batch, 0)

        for j in range(nw_nodes // L):
            d = deg_t[pl.ds(j * L, L)]
            node = jnp.full((L,), base_n + j * L, jnp.int32) + _iota16()
            real = node < N
            deg_t[pl.ds(j * L, L)] = jnp.where(real, _rsqrt16(d), 0.0)
            sw_t[pl.ds(j * L, L)] = jnp.where(real, 1.0 / d, 0.0)
        pltpu.sync_copy(deg_t, dinv_hbm.at[pl.ds(base_n, nw_nodes)])
        pltpu.sync_copy(sw_t, selfw_hbm.at[pl.ds(base_n, nw_nodes)])

    return degree


def _make_norm(Ep):
    ebw = Ep // NW

    @functools.partial(
        pl.kernel,
        out_type=jax.ShapeDtypeStruct((Ep,), jnp.float32),
        mesh=_MESH,
        compiler_params=_SC_PARAMS,
        scratch_types=[
            pltpu.VMEM((NP,), jnp.float32),
            pltpu.VMEM((K,), jnp.int32),
            pltpu.VMEM((K,), jnp.int32),
            pltpu.VMEM((K,), jnp.float32),
        ],
    )
    def norm(src_hbm, dst_hbm, dinv_hbm, nrm_hbm, dinv_v, sbuf, dbuf, fbuf):
        wid = _wid()
        pltpu.sync_copy(dinv_hbm, dinv_v)

        def batch(bi, _):
            a = pl.multiple_of(wid * ebw + bi * K, 8)
            pltpu.sync_copy(src_hbm.at[pl.ds(a, K)], sbuf)
            pltpu.sync_copy(dst_hbm.at[pl.ds(a, K)], dbuf)
            for j in range(K // L):
                si = sbuf[pl.ds(j * L, L)]
                di = dbuf[pl.ds(j * L, L)]
                fbuf[pl.ds(j * L, L)] = (plsc.load_gather(dinv_v, [si])
                                         * plsc.load_gather(dinv_v, [di]))
            pltpu.sync_copy(fbuf, nrm_hbm.at[pl.ds(a, K)])
            return 0
        lax.fori_loop(0, ebw // K, batch, 0)

    return norm


def _make_propagate(D, C, Ep):
    Nc = NP // C
    passes = C // NW

    @functools.partial(
        pl.kernel,
        out_type=jax.ShapeDtypeStruct((NP, D), jnp.float32),
        mesh=_MESH,
        compiler_params=_SC_PARAMS,
        scratch_types=[
            pltpu.VMEM((Nc + 1, D), jnp.float32),
            pltpu.VMEM((KE,), jnp.int32),
            pltpu.VMEM((KE,), jnp.int32),
            pltpu.VMEM((KE,), jnp.float32),
            pltpu.VMEM((KE,), jnp.int32),
            pltpu.VMEM((KE,), jnp.int32),
            pltpu.VMEM((KE,), jnp.float32),
            pltpu.VMEM((KE, D), jnp.float32),
            pltpu.VMEM((KE, D), jnp.float32),
            pltpu.VMEM((Nc,), jnp.float32),
            pltpu.VMEM((EO_LEN,), jnp.int32),
            pltpu.SemaphoreType.DMA,
            pltpu.SemaphoreType.DMA,
        ],
    )
    def propagate(h_hbm, src_hbm, dst_hbm, nrm_hbm, selfw_hbm, eo_hbm, out_hbm,
                  acc, sb0, db0, nb0, sb1, db1, nb1, rows0, rows1,
                  swbuf, eo_v, gsem0, gsem1):
        wid = _wid()
        pltpu.sync_copy(eo_hbm, eo_v)

        def one_pass(p, _):
            chunk = p * NW + wid
            base = chunk * Nc

            pltpu.sync_copy(h_hbm.at[pl.ds(base, Nc)], acc.at[pl.ds(0, Nc)])
            pltpu.sync_copy(selfw_hbm.at[pl.ds(base, Nc)], swbuf)

            def initk(k, _):
                w = _splat_f(swbuf, k)
                for j in range(D // L):
                    acc[k, pl.ds(j * L, L)] = acc[k, pl.ds(j * L, L)] * w
                return 0
            lax.fori_loop(0, Nc, initk, 0)

            my_lo = _scalar_i(eo_v, chunk)
            my_hi = _scalar_i(eo_v, chunk + 1)
            a0 = (my_lo // 8) * 8
            nb = jnp.maximum((my_hi - a0 + KE - 1) // KE, 0)
            nb2 = (nb + 1) // 2

            def addr(bi):
                return pl.multiple_of(
                    a0 + jnp.minimum(bi, (Ep - a0) // KE - 1) * KE, 8)

            def fetch(a, sb, db, nbf, rows_, sem):
                pltpu.sync_copy(src_hbm.at[pl.ds(a, KE)], sb)
                pltpu.sync_copy(dst_hbm.at[pl.ds(a, KE)], db)
                pltpu.sync_copy(nrm_hbm.at[pl.ds(a, KE)], nbf)
                pltpu.async_copy(h_hbm.at[sb], rows_, sem)

            def drain(rows_, sem):
                pltpu.make_async_copy(h_hbm.at[pl.ds(0, KE)], rows_, sem).wait()

            def consume(a, sb, db, nbf, rows_):
                for j in range(KE // L):
                    eid = jnp.full((L,), a + j * L, jnp.int32) + _iota16()
                    ok = (eid >= my_lo) & (eid < my_hi)
                    nv = nbf[pl.ds(j * L, L)]
                    nbf[pl.ds(j * L, L)] = jnp.where(ok, nv, 0.0)
                    dv = db[pl.ds(j * L, L)] - base
                    dv = jnp.minimum(jnp.maximum(dv, 0), Nc)
                    db[pl.ds(j * L, L)] = dv

                def scalek8(i, _):
                    k0 = i * 8
                    ws = [_splat_f(nbf, k0 + c) for c in range(8)]
                    ls = [_scalar_i(db, k0 + c) for c in range(8)]
                    for j in range(D // L):
                        for c in range(8):
                            plsc.addupdate(
                                acc.at[ls[c], pl.ds(j * L, L)],
                                rows_[k0 + c, pl.ds(j * L, L)] * ws[c])
                    return 0
                lax.fori_loop(0, KE // 8, scalek8, 0)

            fetch(addr(0), sb0, db0, nb0, rows0, gsem0)

            def pair(i, _):
                a_odd = addr(2 * i + 1)
                fetch(a_odd, sb1, db1, nb1, rows1, gsem1)
                drain(rows0, gsem0)
                consume(addr(2 * i), sb0, db0, nb0, rows0)
                fetch(addr(2 * i + 2), sb0, db0, nb0, rows0, gsem0)
                drain(rows1, gsem1)
                consume(a_odd, sb1, db1, nb1, rows1)
                return 0
            lax.fori_loop(0, nb2, pair, 0)
            drain(rows0, gsem0)

            pltpu.sync_copy(acc.at[pl.ds(0, Nc)], out_hbm.at[pl.ds(base, Nc)])
            return 0

        lax.fori_loop(0, passes, one_pass, 0)

    return propagate


def _matmul(x, w, b_in=None, b_out=None, block_m=1024):
    m, d_in = x.shape
    d_out = w.shape[1]
    relu_in = b_in is not None
    add_out = b_out is not None

    def kern(x_ref, w_ref, bi_ref, bo_ref, o_ref):
        a = x_ref[...]
        if relu_in:
            a = jnp.maximum(a + bi_ref[...], 0.0)
        acc = jnp.dot(a, w_ref[...], preferred_element_type=jnp.float32)
        if add_out:
            acc = acc + bo_ref[...]
        o_ref[...] = acc

    bi = b_in.reshape(1, d_in) if relu_in else jnp.zeros((1, d_in), jnp.float32)
    bo = b_out.reshape(1, d_out) if add_out else jnp.zeros((1, d_out), jnp.float32)
    return pl.pallas_call(
        kern,
        grid=(m // block_m,),
        in_specs=[
            pl.BlockSpec((block_m, d_in), lambda i: (i, 0)),
            pl.BlockSpec((d_in, d_out), lambda i: (0, 0)),
            pl.BlockSpec((1, d_in), lambda i: (0, 0)),
            pl.BlockSpec((1, d_out), lambda i: (0, 0)),
        ],
        out_specs=pl.BlockSpec((block_m, d_out), lambda i: (i, 0)),
        out_shape=jax.ShapeDtypeStruct((m, d_out), jnp.float32),
    )(x, w, bi, bo)


@jax.jit
def kernel(x, edge_index, W1, b1, W2, b2, W3, b3, W4, b4, W5, b5, Wfc, bfc):
    E = edge_index.shape[1]
    Ep = ((E + NW * K) // (NW * K)) * (NW * K)

    src = edge_index[0].astype(jnp.int32)
    dst = edge_index[1].astype(jnp.int32)
    perm = jnp.argsort(dst)
    src_s = jnp.concatenate([src[perm], jnp.zeros((Ep - E,), jnp.int32)])
    dst_s = jnp.concatenate([dst[perm], jnp.full((Ep - E,), N, jnp.int32)])

    def edge_offsets(C):
        bounds = jnp.array([min(i * (NP // C), N) for i in range(C + 1)]
                           + [0] * (EO_LEN - C - 1), jnp.int32)
        return jnp.searchsorted(dst_s, bounds).astype(jnp.int32)

    eo32 = edge_offsets(NW)
    eo64 = edge_offsets(2 * NW)
    eo128 = edge_offsets(4 * NW)

    dinv, selfw = _make_degree()(dst_s, eo32)
    nrm = _make_norm(Ep)(src_s, dst_s, dinv)

    xp = jnp.pad(x, ((0, NP - N), (0, 0)))
    W5p = jnp.pad(W5, ((0, 0), (0, 118)))
    b5p = jnp.pad(b5, (0, 118))
    Wfcp = jnp.pad(Wfc, ((0, 118), (0, 125)))
    bfcp = jnp.pad(bfc, (0, 125))

    prop512 = _make_propagate(512, 4 * NW, Ep)
    prop256 = _make_propagate(256, 2 * NW, Ep)
    prop128 = _make_propagate(128, NW, Ep)

    h = _matmul(xp, W1)
    p = prop512(h, src_s, dst_s, nrm, selfw, eo128)
    h = _matmul(p, W2, b_in=b1)
    p = prop512(h, src_s, dst_s, nrm, selfw, eo128)
    h = _matmul(p, W3, b_in=b2)
    p = prop256(h, src_s, dst_s, nrm, selfw, eo64)
    h = _matmul(p, W4, b_in=b3)
    p = prop128(h, src_s, dst_s, nrm, selfw, eo32)
    h = _matmul(p, W5p, b_in=b4)
    p = prop128(h, src_s, dst_s, nrm, selfw, eo32)
    out = _matmul(p, Wfcp, b_in=b5p, b_out=bfcp)
    return out[:N, :3]


# speedup vs baseline: 3.8272x; 1.1401x over previous
import functools

import jax
import jax.numpy as jnp
from jax import lax
from jax.experimental import pallas as pl
from jax.experimental.pallas import tpu as pltpu
from jax.experimental.pallas import tpu_sc as plsc

NC = 2
NS = 16
NW = NC * NS
L = 16

N = 10000
NP = 10240
K = 128
KE = 64
EO_LEN = 136

_MESH = plsc.VectorSubcoreMesh(core_axis_name="c", subcore_axis_name="s")
_SC_PARAMS = pltpu.CompilerParams(needs_layout_passes=False)


def _splat_f(buf, k):
    return plsc.load_gather(buf, [jnp.full((L,), k, jnp.int32)])


def _scalar_i(buf, i):
    return jnp.max(plsc.load_gather(buf, [jnp.full((L,), i, jnp.int32)]))


def _rsqrt16(x):
    i = lax.bitcast_convert_type(x, jnp.int32)
    i = jnp.int32(0x5F3759DF) - (i >> 1)
    y = lax.bitcast_convert_type(i, jnp.float32)
    for _ in range(3):
        y = y * (1.5 - 0.5 * x * y * y)
    return y


def _iota16():
    return lax.iota(jnp.int32, L)


def _wid():
    return lax.axis_index("s") * NC + lax.axis_index("c")


def _make_degree():
    nw_nodes = NP // NW

    @functools.partial(
        pl.kernel,
        out_type=(
            jax.ShapeDtypeStruct((NP,), jnp.float32),
            jax.ShapeDtypeStruct((NP,), jnp.float32),
        ),
        mesh=_MESH,
        compiler_params=_SC_PARAMS,
        scratch_types=[
            pltpu.VMEM((nw_nodes,), jnp.float32),
            pltpu.VMEM((nw_nodes,), jnp.float32),
            pltpu.VMEM((K,), jnp.int32),
            pltpu.VMEM((EO_LEN,), jnp.int32),
        ],
    )
    def degree(dst_hbm, eo_hbm, dinv_hbm, selfw_hbm, deg_t, sw_t, dbuf, eo_v):
        wid = _wid()
        base_n = wid * nw_nodes
        pltpu.sync_copy(eo_hbm, eo_v)

        for j in range(nw_nodes // L):
            deg_t[pl.ds(j * L, L)] = jnp.full((L,), 1.0, jnp.float32)

        my_lo = _scalar_i(eo_v, wid)
        my_hi = _scalar_i(eo_v, wid + 1)
        a0 = (my_lo // 8) * 8
        nb = jnp.maximum((my_hi - a0 + K - 1) // K, 0)

        def batch(bi, _):
            a = pl.multiple_of(a0 + bi * K, 8)
            pltpu.sync_copy(dst_hbm.at[pl.ds(a, K)], dbuf)

            def onehot(k, _):
                eid = a + k
                valid = (eid >= my_lo) & (eid < my_hi)
                lidx = _scalar_i(dbuf, k) - base_n
                lidx = jnp.minimum(jnp.maximum(lidx, 0), nw_nodes - 1)
                row = (lidx // L) * L
                inc = jnp.where((_iota16() == lidx - row)
                                & jnp.full((L,), valid, jnp.bool_), 1.0, 0.0)
                deg_t[pl.ds(row, L)] = deg_t[pl.ds(row, L)] + inc
                return 0
            lax.fori_loop(0, K, onehot, 0)
            return 0
        lax.fori_loop(0, nb, batch, 0)

        for j in range(nw_nodes // L):
            d = deg_t[pl.ds(j * L, L)]
            node = jnp.full((L,), base_n + j * L, jnp.int32) + _iota16()
            real = node < N
            deg_t[pl.ds(j * L, L)] = jnp.where(real, _rsqrt16(d), 0.0)
            sw_t[pl.ds(j * L, L)] = jnp.where(real, 1.0 / d, 0.0)
        pltpu.sync_copy(deg_t, dinv_hbm.at[pl.ds(base_n, nw_nodes)])
        pltpu.sync_copy(sw_t, selfw_hbm.at[pl.ds(base_n, nw_nodes)])

    return degree


def _make_norm(Ep):
    ebw = Ep // NW

    @functools.partial(
        pl.kernel,
        out_type=jax.ShapeDtypeStruct((Ep,), jnp.float32),
        mesh=_MESH,
        compiler_params=_SC_PARAMS,
        scratch_types=[
            pltpu.VMEM((NP,), jnp.float32),
            pltpu.VMEM((K,), jnp.int32),
            pltpu.VMEM((K,), jnp.int32),
            pltpu.VMEM((K,), jnp.float32),
        ],
    )
    def norm(src_hbm, dst_hbm, dinv_hbm, nrm_hbm, dinv_v, sbuf, dbuf, fbuf):
        wid = _wid()
        pltpu.sync_copy(dinv_hbm, dinv_v)

        def batch(bi, _):
            a = pl.multiple_of(wid * ebw + bi * K, 8)
            pltpu.sync_copy(src_hbm.at[pl.ds(a, K)], sbuf)
            pltpu.sync_copy(dst_hbm.at[pl.ds(a, K)], dbuf)
            for j in range(K // L):
                si = sbuf[pl.ds(j * L, L)]
                di = dbuf[pl.ds(j * L, L)]
                fbuf[pl.ds(j * L, L)] = (plsc.load_gather(dinv_v, [si])
                                         * plsc.load_gather(dinv_v, [di]))
            pltpu.sync_copy(fbuf, nrm_hbm.at[pl.ds(a, K)])
            return 0
        lax.fori_loop(0, ebw // K, batch, 0)

    return norm


def _unpack16(v):
    lo = lax.bitcast_convert_type(lax.shift_left(v, 16), jnp.float32)
    hi = lax.bitcast_convert_type(v & jnp.int32(-65536), jnp.float32)
    return lo, hi


def _make_propagate(D, C, Ep, packed=False):
    Nc = NP // C
    passes = C // NW
    D2 = D // 2
    row_w = D2 if packed else D
    row_t = jnp.int32 if packed else jnp.float32

    @functools.partial(
        pl.kernel,
        out_type=jax.ShapeDtypeStruct((NP, D), jnp.float32),
        mesh=_MESH,
        compiler_params=_SC_PARAMS,
        scratch_types=[
            pltpu.VMEM((Nc + 1, D), jnp.float32),
            pltpu.VMEM((KE,), jnp.int32),
            pltpu.VMEM((KE,), jnp.int32),
            pltpu.VMEM((KE,), jnp.float32),
            pltpu.VMEM((KE,), jnp.int32),
            pltpu.VMEM((KE,), jnp.int32),
            pltpu.VMEM((KE,), jnp.float32),
            pltpu.VMEM((KE, row_w), row_t),
            pltpu.VMEM((KE, row_w), row_t),
            pltpu.VMEM((Nc if packed else 1, row_w), row_t),

            pltpu.VMEM((Nc,), jnp.float32),
            pltpu.VMEM((EO_LEN,), jnp.int32),
            pltpu.SemaphoreType.DMA,
            pltpu.SemaphoreType.DMA,
        ],
    )
    def propagate(h_hbm, src_hbm, dst_hbm, nrm_hbm, selfw_hbm, eo_hbm, out_hbm,
                  acc, sb0, db0, nb0, sb1, db1, nb1, rows0, rows1,
                  hstage, swbuf, eo_v, gsem0, gsem1):
        wid = _wid()
        pltpu.sync_copy(eo_hbm, eo_v)

        def one_pass(p, _):
            chunk = p * NW + wid
            base = chunk * Nc

            pltpu.sync_copy(selfw_hbm.at[pl.ds(base, Nc)], swbuf)
            if packed:
                pltpu.sync_copy(h_hbm.at[pl.ds(base, Nc)], hstage)

                def initk(k, _):
                    w = _splat_f(swbuf, k)
                    for j in range(D2 // L):
                        lo, hi = _unpack16(hstage[k, pl.ds(j * L, L)])
                        acc[k, pl.ds(j * L, L)] = lo * w
                        acc[k, pl.ds(D2 + j * L, L)] = hi * w
                    return 0
            else:
                pltpu.sync_copy(h_hbm.at[pl.ds(base, Nc)],
                                acc.at[pl.ds(0, Nc)])

                def initk(k, _):
                    w = _splat_f(swbuf, k)
                    for j in range(D // L):
                        acc[k, pl.ds(j * L, L)] = acc[k, pl.ds(j * L, L)] * w
                    return 0
            lax.fori_loop(0, Nc, initk, 0)

            my_lo = _scalar_i(eo_v, chunk)
            my_hi = _scalar_i(eo_v, chunk + 1)
            a0 = (my_lo // 8) * 8
            nb = jnp.maximum((my_hi - a0 + KE - 1) // KE, 0)
            nb2 = (nb + 1) // 2

            def addr(bi):
                return pl.multiple_of(
                    a0 + jnp.minimum(bi, (Ep - a0) // KE - 1) * KE, 8)

            def fetch(a, sb, db, nbf, rows_, sem):
                pltpu.sync_copy(src_hbm.at[pl.ds(a, KE)], sb)
                pltpu.sync_copy(dst_hbm.at[pl.ds(a, KE)], db)
                pltpu.sync_copy(nrm_hbm.at[pl.ds(a, KE)], nbf)
                pltpu.async_copy(h_hbm.at[sb], rows_, sem)

            def drain(rows_, sem):
                pltpu.make_async_copy(h_hbm.at[pl.ds(0, KE)], rows_, sem).wait()

            def consume(a, sb, db, nbf, rows_):
                for j in range(KE // L):
                    eid = jnp.full((L,), a + j * L, jnp.int32) + _iota16()
                    ok = (eid >= my_lo) & (eid < my_hi)
                    nv = nbf[pl.ds(j * L, L)]
                    nbf[pl.ds(j * L, L)] = jnp.where(ok, nv, 0.0)
                    dv = db[pl.ds(j * L, L)] - base
                    dv = jnp.minimum(jnp.maximum(dv, 0), Nc)
                    db[pl.ds(j * L, L)] = dv

                def scalek8(i, _):
                    k0 = i * 8
                    ws = [_splat_f(nbf, k0 + c) for c in range(8)]
                    ls = [_scalar_i(db, k0 + c) for c in range(8)]
                    if packed:
                        for j in range(D2 // L):
                            for c in range(8):
                                lo, hi = _unpack16(
                                    rows_[k0 + c, pl.ds(j * L, L)])
                                plsc.addupdate(
                                    acc.at[ls[c], pl.ds(j * L, L)],
                                    lo * ws[c])
                                plsc.addupdate(
                                    acc.at[ls[c], pl.ds(D2 + j * L, L)],
                                    hi * ws[c])
                    else:
                        for j in range(D // L):
                            for c in range(8):
                                plsc.addupdate(
                                    acc.at[ls[c], pl.ds(j * L, L)],
                                    rows_[k0 + c, pl.ds(j * L, L)] * ws[c])
                    return 0
                lax.fori_loop(0, KE // 8, scalek8, 0)

            fetch(addr(0), sb0, db0, nb0, rows0, gsem0)

            def pair(i, _):
                a_odd = addr(2 * i + 1)
                fetch(a_odd, sb1, db1, nb1, rows1, gsem1)
                drain(rows0, gsem0)
                consume(addr(2 * i), sb0, db0, nb0, rows0)
                fetch(addr(2 * i + 2), sb0, db0, nb0, rows0, gsem0)
                drain(rows1, gsem1)
                consume(a_odd, sb1, db1, nb1, rows1)
                return 0
            lax.fori_loop(0, nb2, pair, 0)
            drain(rows0, gsem0)

            pltpu.sync_copy(acc.at[pl.ds(0, Nc)], out_hbm.at[pl.ds(base, Nc)])
            return 0

        lax.fori_loop(0, passes, one_pass, 0)

    return propagate


def _matmul(x, w, b_in=None, b_out=None, block_m=1024,
            out_dtype=jnp.float32):
    m, d_in = x.shape
    d_out = w.shape[1]
    relu_in = b_in is not None
    add_out = b_out is not None

    def kern(x_ref, w_ref, bi_ref, bo_ref, o_ref):
        a = x_ref[...]
        if relu_in:
            a = jnp.maximum(a + bi_ref[...], 0.0)
        acc = jnp.dot(a, w_ref[...], preferred_element_type=jnp.float32)
        if add_out:
            acc = acc + bo_ref[...]
        o_ref[...] = acc.astype(out_dtype)

    bi = b_in.reshape(1, d_in) if relu_in else jnp.zeros((1, d_in), jnp.float32)
    bo = b_out.reshape(1, d_out) if add_out else jnp.zeros((1, d_out), jnp.float32)
    return pl.pallas_call(
        kern,
        grid=(m // block_m,),
        in_specs=[
            pl.BlockSpec((block_m, d_in), lambda i: (i, 0)),
            pl.BlockSpec((d_in, d_out), lambda i: (0, 0)),
            pl.BlockSpec((1, d_in), lambda i: (0, 0)),
            pl.BlockSpec((1, d_out), lambda i: (0, 0)),
        ],
        out_specs=pl.BlockSpec((block_m, d_out), lambda i: (i, 0)),
        out_shape=jax.ShapeDtypeStruct((m, d_out), out_dtype),
    )(x, w, bi, bo)


def _pack_bf16(h):
    return lax.bitcast_convert_type(
        h.reshape(h.shape[0], h.shape[1] // 2, 2), jnp.int32)


@jax.jit
def kernel(x, edge_index, W1, b1, W2, b2, W3, b3, W4, b4, W5, b5, Wfc, bfc):
    E = edge_index.shape[1]
    Ep = ((E + NW * K) // (NW * K)) * (NW * K)

    src = edge_index[0].astype(jnp.int32)
    dst = edge_index[1].astype(jnp.int32)
    perm = jnp.argsort(dst)
    src_s = jnp.concatenate([src[perm], jnp.zeros((Ep - E,), jnp.int32)])
    dst_s = jnp.concatenate([dst[perm], jnp.full((Ep - E,), N, jnp.int32)])

    def edge_offsets(C):
        bounds = jnp.array([min(i * (NP // C), N) for i in range(C + 1)]
                           + [0] * (EO_LEN - C - 1), jnp.int32)
        return jnp.searchsorted(dst_s, bounds).astype(jnp.int32)

    eo32 = edge_offsets(NW)
    eo64 = edge_offsets(2 * NW)
    eo128 = edge_offsets(4 * NW)

    dinv, selfw = _make_degree()(dst_s, eo32)
    nrm = _make_norm(Ep)(src_s, dst_s, dinv)

    xp = jnp.pad(x, ((0, NP - N), (0, 0)))
    W5p = jnp.pad(W5, ((0, 0), (0, 118)))
    b5p = jnp.pad(b5, (0, 118))
    Wfcp = jnp.pad(Wfc, ((0, 118), (0, 125)))
    bfcp = jnp.pad(bfc, (0, 125))

    prop512 = _make_propagate(512, 4 * NW, Ep, packed=True)
    prop256 = _make_propagate(256, 2 * NW, Ep, packed=True)
    prop128 = _make_propagate(128, NW, Ep)

    pm512 = jnp.concatenate([jnp.arange(0, 512, 2), jnp.arange(1, 512, 2)])
    pm256 = jnp.concatenate([jnp.arange(0, 256, 2), jnp.arange(1, 256, 2)])

    bf = jnp.bfloat16
    h = _matmul(xp, W1, out_dtype=bf)
    p = prop512(_pack_bf16(h), src_s, dst_s, nrm, selfw, eo128)
    h = _matmul(p, W2[pm512], b_in=b1[pm512], out_dtype=bf)
    p = prop512(_pack_bf16(h), src_s, dst_s, nrm, selfw, eo128)
    h = _matmul(p, W3[pm512], b_in=b2[pm512], out_dtype=bf)
    p = prop256(_pack_bf16(h), src_s, dst_s, nrm, selfw, eo64)
    h = _matmul(p, W4[pm256], b_in=b3[pm256])
    p = prop128(h, src_s, dst_s, nrm, selfw, eo32)
    h = _matmul(p, W5p, b_in=b4)
    p = prop128(h, src_s, dst_s, nrm, selfw, eo32)
    out = _matmul(p, Wfcp, b_in=b5p, b_out=bfcp)
    return out[:N, :3]
